# baseline (speedup 1.0000x reference)
import functools

import jax
import jax.numpy as jnp
from jax import lax
from jax.experimental import pallas as pl
from jax.experimental.pallas import tpu as pltpu
from jax.experimental.pallas import tpu_sc as plsc

_K0 = 928981903
_K1 = 3453687069

_NC = 2
_NS = 16
_NW = _NC * _NS
_GRP = 128


def _keep_scale(idx_i32):
  k0 = jnp.uint32(_K0)
  k1 = jnp.uint32(_K1)
  k2 = k0 ^ k1 ^ jnp.uint32(0x1BD11BDA)
  ks = (k0, k1, k2)
  rot = ((13, 15, 26, 6), (17, 29, 16, 24))

  x0 = jnp.zeros((16,), jnp.uint32) + k0
  x1 = idx_i32.astype(jnp.uint32) + k1
  for i in range(5):
    for r in rot[i % 2]:
      x0 = x0 + x1
      x1 = (x1 << jnp.uint32(r)) | (x1 >> jnp.uint32(32 - r))
      x1 = x1 ^ x0
    x0 = x0 + ks[(i + 1) % 3]
    x1 = x1 + ks[(i + 2) % 3] + jnp.uint32(i + 1)
  bits = x0 ^ x1
  keep = (bits >> jnp.uint32(9)) <= jnp.uint32(6710886)
  return jnp.where(keep, jnp.float32(1.25), jnp.float32(0.0))


@functools.lru_cache(maxsize=None)
def _make_sc_lookup(n_flat, v, d):
  assert d == 32 and n_flat % (_NW * _GRP) == 0
  gpw = n_flat // (_NW * _GRP)
  mesh = plsc.VectorSubcoreMesh(core_axis_name="c", subcore_axis_name="s")

  @functools.partial(
      pl.kernel,
      mesh=mesh,
      out_type=jax.ShapeDtypeStruct((n_flat, d), jnp.float32),
      compiler_params=pltpu.CompilerParams(use_tc_tiling_on_sc=False),
      scratch_types=[
          pltpu.VMEM((gpw * _GRP,), jnp.int32),
          pltpu.VMEM((_GRP, d), jnp.float32),
          pltpu.VMEM((_GRP,), jnp.float32),
          pltpu.SemaphoreType.DMA,
      ],
  )
  def lookup(x_hbm, w_hbm, out_hbm, idx_v, buf_v, scale_v, sem):
    wid = lax.axis_index("s") * _NC + lax.axis_index("c")
    g0 = wid * gpw
    pltpu.sync_copy(x_hbm.at[pl.ds(g0 * _GRP, gpw * _GRP)], idx_v)

    def per_group(g, carry):
      cp = pltpu.async_copy(
          w_hbm.at[idx_v.at[pl.ds(g * _GRP, _GRP)]], buf_v, sem)
      for t in range(_GRP // 16):
        scale_v[pl.ds(t * 16, 16)] = _keep_scale(
            idx_v[pl.ds(g * _GRP + t * 16, 16)])
      cp.wait()

      def blk16(rr, c):
        base = rr * 16
        sv = scale_v[pl.ds(base, 16)]
        for k in range(16):
          r = base + k
          s = sv[k]
          buf_v[r, pl.ds(0, 16)] = buf_v[r, pl.ds(0, 16)] * s
          buf_v[r, pl.ds(16, 16)] = buf_v[r, pl.ds(16, 16)] * s
        return c

      lax.fori_loop(0, _GRP // 16, blk16, 0)
      pltpu.sync_copy(buf_v, out_hbm.at[pl.ds((g0 + g) * _GRP, _GRP)])
      return carry

    lax.fori_loop(0, gpw, per_group, 0)

  return lookup


def kernel(x, W):
  b, l = x.shape
  v, d = W.shape
  n_flat = b * l
  x2 = x.reshape(n_flat)
  out = _make_sc_lookup(n_flat, v, d)(x2, W)
  return out.reshape(b, l, d)


# speedup vs baseline: 1.0462x; 1.0462x over previous
import functools

import jax
import jax.numpy as jnp
from jax import lax
from jax.experimental import pallas as pl
from jax.experimental.pallas import tpu as pltpu
from jax.experimental.pallas import tpu_sc as plsc

_K0 = 928981903
_K1 = 3453687069

_NC = 2
_NS = 16
_NW = _NC * _NS
_GRP = 128


def _keep_scale(idx_i32):
  k0 = jnp.uint32(_K0)
  k1 = jnp.uint32(_K1)
  k2 = k0 ^ k1 ^ jnp.uint32(0x1BD11BDA)
  ks = (k0, k1, k2)
  rot = ((13, 15, 26, 6), (17, 29, 16, 24))

  x0 = jnp.zeros((16,), jnp.uint32) + k0
  x1 = idx_i32.astype(jnp.uint32) + k1
  for i in range(5):
    for r in rot[i % 2]:
      x0 = x0 + x1
      x1 = (x1 << jnp.uint32(r)) | (x1 >> jnp.uint32(32 - r))
      x1 = x1 ^ x0
    x0 = x0 + ks[(i + 1) % 3]
    x1 = x1 + ks[(i + 2) % 3] + jnp.uint32(i + 1)
  bits = x0 ^ x1
  keep = (bits >> jnp.uint32(9)) <= jnp.uint32(6710886)
  return jnp.where(keep, jnp.float32(1.25), jnp.float32(0.0))


_NBUF = 5


@functools.lru_cache(maxsize=None)
def _make_sc_lookup(n_flat, v, d):
  assert d == 32 and n_flat % (_NW * _GRP) == 0
  gpw = n_flat // (_NW * _GRP)
  assert gpw % _NBUF == 0
  kmax = gpw // _NBUF
  npw = gpw * _GRP
  mesh = plsc.VectorSubcoreMesh(core_axis_name="c", subcore_axis_name="s")

  @functools.partial(
      pl.kernel,
      mesh=mesh,
      out_type=jax.ShapeDtypeStruct((n_flat, d), jnp.float32),
      compiler_params=pltpu.CompilerParams(use_tc_tiling_on_sc=False),
      scratch_types=[
          pltpu.VMEM((npw,), jnp.int32),
          pltpu.VMEM((npw,), jnp.float32),
          pltpu.VMEM((_NBUF, _GRP, d), jnp.float32),
          pltpu.VMEM((_NBUF, _GRP, d), jnp.float32),
          pltpu.SemaphoreType.DMA((_NBUF,)),
          pltpu.SemaphoreType.DMA((_NBUF,)),
      ],
  )
  def lookup(x_hbm, w_hbm, out_hbm, idx_v, scale_v, ibuf, obuf, gsem, ssem):
    wid = lax.axis_index("s") * _NC + lax.axis_index("c")
    g0 = wid * gpw
    pltpu.sync_copy(x_hbm.at[pl.ds(g0 * _GRP, npw)], idx_v)

    for b in range(_NBUF):
      pltpu.async_copy(
          w_hbm.at[idx_v.at[pl.ds(b * _GRP, _GRP)]], ibuf.at[b], gsem.at[b])

    def scales(t, c):
      scale_v[pl.ds(t * 16, 16)] = _keep_scale(idx_v[pl.ds(t * 16, 16)])
      return c

    lax.fori_loop(0, npw // 16, scales, 0)

    def step(k, carry):
      for b in range(_NBUF):
        g = k * _NBUF + b
        pltpu.make_async_copy(
            w_hbm.at[idx_v.at[pl.ds(g * _GRP, _GRP)]], ibuf.at[b],
            gsem.at[b]).wait()
        @pl.when(k > 0)
        def _():
          pltpu.make_async_copy(
              obuf.at[b], out_hbm.at[pl.ds((g0 + g) * _GRP, _GRP)],
              ssem.at[b]).wait()

        def blk16(rr, c):
          base = rr * 16
          sv = scale_v[pl.ds(g * _GRP + base, 16)]
          for j in range(16):
            r = base + j
            s = sv[j]
            obuf[b, r, pl.ds(0, 16)] = ibuf[b, r, pl.ds(0, 16)] * s
            obuf[b, r, pl.ds(16, 16)] = ibuf[b, r, pl.ds(16, 16)] * s
          return c

        lax.fori_loop(0, _GRP // 16, blk16, 0)
        pltpu.async_copy(
            obuf.at[b], out_hbm.at[pl.ds((g0 + g) * _GRP, _GRP)], ssem.at[b])

        @pl.when(k < kmax - 1)
        def _():
          pltpu.async_copy(
              w_hbm.at[idx_v.at[pl.ds((g + _NBUF) * _GRP, _GRP)]], ibuf.at[b],
              gsem.at[b])

      return carry

    lax.fori_loop(0, kmax, step, 0)

    for b in range(_NBUF):
      pltpu.make_async_copy(
          obuf.at[b], out_hbm.at[pl.ds(g0 * _GRP, _GRP)], ssem.at[b]).wait()

  return lookup


def kernel(x, W):
  b, l = x.shape
  v, d = W.shape
  n_flat = b * l
  x2 = x.reshape(n_flat)
  out = _make_sc_lookup(n_flat, v, d)(x2, W)
  return out.reshape(b, l, d)
